# Initial kernel scaffold; baseline (speedup 1.0000x reference)
import functools

import numpy as np
import jax
import jax.numpy as jnp
from jax import lax
from jax.experimental import pallas as pl
from jax.experimental.pallas import tpu as pltpu
from jax.experimental.pallas import tpu_sc as plsc

NUM_FIELDS = 26
VOCAB = 100000
DIM = 32
BATCH = 4096
HIST = 50

LANES = 16
IDS = NUM_FIELDS * HIST
IDS_PAD = 1312
NCHUNK = IDS_PAD // LANES
NW = 32
B_PER_W = BATCH // NW
HALF = DIM // 2

_FID_NP = np.concatenate([
    np.repeat(np.arange(NUM_FIELDS, dtype=np.int32), HIST),
    np.full((IDS_PAD - IDS,), 31, dtype=np.int32),
])
_OFF_NP = np.where(_FID_NP < NUM_FIELDS, _FID_NP * VOCAB, 0).astype(np.int32)


def _body(x_hbm, tab_hbm, fid_hbm, off_hbm, out_hbm,
          xv, gidv, fidv, offv, rows, cnt, recip, outv, sem):
    wid = lax.axis_index("s") * 2 + lax.axis_index("c")
    base = wid * B_PER_W
    zero16 = jnp.zeros((LANES,), jnp.float32)

    pltpu.sync_copy(fid_hbm, fidv)
    pltpu.sync_copy(off_hbm, offv)
    xv[pl.ds(IDS_PAD - LANES, LANES)] = jnp.zeros((LANES,), jnp.int32)

    @pl.loop(0, B_PER_W)
    def _batch(b):
        bb = base + b
        pltpu.sync_copy(x_hbm.at[bb], xv.at[pl.ds(0, IDS)])

        cnt[pl.ds(0, LANES)] = zero16
        cnt[pl.ds(LANES, LANES)] = zero16

        for c in range(NCHUNK):
            sl = pl.ds(c * LANES, LANES)
            xc = xv[sl]
            gidv[sl] = xc + offv[sl]
            m = jnp.where(xc != 0, 1.0, 0.0).astype(jnp.float32)
            plsc.addupdate_scatter(cnt, [fidv[sl]], m)

        pltpu.async_copy(tab_hbm.at[gidv], rows, sem).wait()

        recip[pl.ds(0, LANES)] = 1.0 / jnp.maximum(cnt[pl.ds(0, LANES)], 1.0)
        recip[pl.ds(LANES, LANES)] = 1.0 / jnp.maximum(
            cnt[pl.ds(LANES, LANES)], 1.0)

        for f in range(NUM_FIELDS):
            rf = plsc.load_gather(
                recip, [jnp.full((LANES,), f, jnp.int32)])

            @pl.loop(0, HIST, init_carry=(zero16, zero16), unroll=5)
            def _sum(l, carry):
                a0, a1 = carry
                r = f * HIST + l
                a0 = a0 + rows[r, pl.ds(0, HALF)]
                a1 = a1 + rows[r, pl.ds(HALF, HALF)]
                return a0, a1

            a0, a1 = _sum
            outv[f, pl.ds(0, HALF)] = a0 * rf
            outv[f, pl.ds(HALF, HALF)] = a1 * rf

        pltpu.sync_copy(outv, out_hbm.at[bb])


@jax.jit
def kernel(x, tables):
    x2 = x.reshape(BATCH, IDS)
    tabf = tables.reshape(NUM_FIELDS * VOCAB, DIM)
    fid = jnp.asarray(_FID_NP)
    off = jnp.asarray(_OFF_NP)

    call = pl.kernel(
        _body,
        out_type=jax.ShapeDtypeStruct((BATCH, NUM_FIELDS, DIM), jnp.float32),
        mesh=plsc.VectorSubcoreMesh(core_axis_name="c", subcore_axis_name="s"),
        scratch_types=[
            pltpu.VMEM((IDS_PAD,), jnp.int32),
            pltpu.VMEM((IDS_PAD,), jnp.int32),
            pltpu.VMEM((IDS_PAD,), jnp.int32),
            pltpu.VMEM((IDS_PAD,), jnp.int32),
            pltpu.VMEM((IDS_PAD, DIM), jnp.float32),
            pltpu.VMEM((2 * LANES,), jnp.float32),
            pltpu.VMEM((2 * LANES,), jnp.float32),
            pltpu.VMEM((NUM_FIELDS, DIM), jnp.float32),
            pltpu.SemaphoreType.DMA,
        ],
    )
    out = call(x2, tabf, fid, off)
    return out.reshape(BATCH, NUM_FIELDS * DIM)


# speedup vs baseline: 11.0316x; 11.0316x over previous
import functools

import numpy as np
import jax
import jax.numpy as jnp
from jax import lax
from jax.experimental import pallas as pl
from jax.experimental.pallas import tpu as pltpu
from jax.experimental.pallas import tpu_sc as plsc

NUM_FIELDS = 26
VOCAB = 100000
DIM = 32
BATCH = 4096
HIST = 50

LANES = 16
IDS = NUM_FIELDS * HIST
IDS_PAD = 1312
NCHUNK = IDS_PAD // LANES
NW = 32
B_PER_W = BATCH // NW
HALF = DIM // 2

_FID_NP = np.concatenate([
    np.repeat(np.arange(NUM_FIELDS, dtype=np.int32), HIST),
    np.full((IDS_PAD - IDS,), 31, dtype=np.int32),
])
_OFF_NP = np.where(_FID_NP < NUM_FIELDS, _FID_NP * VOCAB, 0).astype(np.int32)


def _body(x_hbm, tab_hbm, fid_hbm, off_hbm, out_hbm,
          xv, gidv, fidv, offv, rows, cnt, recip, outv, sem):
    wid = lax.axis_index("s") * 2 + lax.axis_index("c")
    base = wid * B_PER_W
    zero16 = jnp.zeros((LANES,), jnp.float32)

    pltpu.sync_copy(fid_hbm, fidv)
    pltpu.sync_copy(off_hbm, offv)
    xv[pl.ds(IDS_PAD - LANES, LANES)] = jnp.zeros((LANES,), jnp.int32)

    @pl.loop(0, B_PER_W)
    def _batch(b):
        bb = base + b
        pltpu.sync_copy(x_hbm.at[bb], xv.at[pl.ds(0, IDS)])

        cnt[pl.ds(0, LANES)] = zero16
        cnt[pl.ds(LANES, LANES)] = zero16

        for c in range(NCHUNK):
            sl = pl.ds(c * LANES, LANES)
            xc = xv[sl]
            gidv[sl] = xc + offv[sl]
            m = jnp.where(xc != 0, 1.0, 0.0).astype(jnp.float32)
            plsc.addupdate_scatter(cnt, [fidv[sl]], m)

        pltpu.async_copy(tab_hbm.at[gidv], rows, sem).wait()

        recip[pl.ds(0, LANES)] = 1.0 / jnp.maximum(cnt[pl.ds(0, LANES)], 1.0)
        recip[pl.ds(LANES, LANES)] = 1.0 / jnp.maximum(
            cnt[pl.ds(LANES, LANES)], 1.0)

        for f in range(NUM_FIELDS):
            rf = plsc.load_gather(
                recip, [jnp.full((LANES,), f, jnp.int32)])

            @pl.loop(0, HIST, init_carry=(zero16, zero16), unroll=5)
            def _sum(l, carry):
                a0, a1 = carry
                r = f * HIST + l
                a0 = a0 + rows[r, pl.ds(0, HALF)]
                a1 = a1 + rows[r, pl.ds(HALF, HALF)]
                return a0, a1

            a0, a1 = _sum
            outv[f, pl.ds(0, HALF)] = a0 * rf
            outv[f, pl.ds(HALF, HALF)] = a1 * rf

        pltpu.sync_copy(outv, out_hbm.at[bb])


@jax.jit
def kernel(x, tables):
    x2 = x.reshape(BATCH, IDS)
    tabf = tables.reshape(NUM_FIELDS * VOCAB, DIM)
    fid = jnp.asarray(_FID_NP)
    off = jnp.asarray(_OFF_NP)

    call = pl.kernel(
        _body,
        out_type=jax.ShapeDtypeStruct((BATCH, NUM_FIELDS, DIM), jnp.float32),
        mesh=plsc.VectorSubcoreMesh(core_axis_name="c", subcore_axis_name="s"),
        compiler_params=pltpu.CompilerParams(
            needs_layout_passes=False, use_tc_tiling_on_sc=False),
        scratch_types=[
            pltpu.VMEM((IDS_PAD,), jnp.int32),
            pltpu.VMEM((IDS_PAD,), jnp.int32),
            pltpu.VMEM((IDS_PAD,), jnp.int32),
            pltpu.VMEM((IDS_PAD,), jnp.int32),
            pltpu.VMEM((IDS_PAD, DIM), jnp.float32),
            pltpu.VMEM((2 * LANES,), jnp.float32),
            pltpu.VMEM((2 * LANES,), jnp.float32),
            pltpu.VMEM((NUM_FIELDS, DIM), jnp.float32),
            pltpu.SemaphoreType.DMA,
        ],
    )
    out = call(x2, tabf, fid, off)
    return out.reshape(BATCH, NUM_FIELDS * DIM)


# speedup vs baseline: 11.2122x; 1.0164x over previous
import functools

import numpy as np
import jax
import jax.numpy as jnp
from jax import lax
from jax.experimental import pallas as pl
from jax.experimental.pallas import tpu as pltpu
from jax.experimental.pallas import tpu_sc as plsc

NUM_FIELDS = 26
VOCAB = 100000
DIM = 32
BATCH = 4096
HIST = 50

LANES = 16
IDS = NUM_FIELDS * HIST
IDS_PAD = 1312
NW = 32
B_PER_W = BATCH // NW
HALF = DIM // 2
OUT_D = NUM_FIELDS * DIM


def _body(x_hbm, tab_hbm, out_hbm,
          xv, gidv, rows, recip2, outv, sem):
    wid = lax.axis_index("s") * 2 + lax.axis_index("c")
    base = wid * B_PER_W
    zero16 = jnp.zeros((LANES,), jnp.float32)
    tail2 = lax.iota(jnp.int32, LANES) >= (LANES - 2)

    gidv[pl.ds(IDS_PAD - LANES, LANES)] = jnp.zeros((LANES,), jnp.int32)

    @pl.loop(0, B_PER_W)
    def _batch(b):
        bb = base + b
        pltpu.sync_copy(x_hbm.at[bb], xv)

        for f in range(NUM_FIELDS):
            off = jnp.full((LANES,), f * VOCAB, jnp.int32)
            xa = xv[f, pl.ds(0, LANES)]
            xb = xv[f, pl.ds(LANES, LANES)]
            xc = xv[f, pl.ds(2 * LANES, LANES)]
            xd = xv[f, pl.ds(HIST - LANES, LANES)]
            gidv[pl.ds(f * HIST, LANES)] = xa + off
            gidv[pl.ds(f * HIST + LANES, LANES)] = xb + off
            gidv[pl.ds(f * HIST + 2 * LANES, LANES)] = xc + off
            gidv[pl.ds(f * HIST + HIST - LANES, LANES)] = xd + off
            cnt = (plsc.all_reduce_population_count(xa != 0)
                   + plsc.all_reduce_population_count(xb != 0)
                   + plsc.all_reduce_population_count(xc != 0)
                   + plsc.all_reduce_population_count((xd != 0) & tail2))
            recip2[f, pl.ds(0, LANES)] = 1.0 / jnp.maximum(
                cnt.astype(jnp.float32), 1.0)

        pltpu.async_copy(tab_hbm.at[gidv], rows, sem).wait()

        for f in range(NUM_FIELDS):
            rf = recip2[f, pl.ds(0, LANES)]

            @pl.loop(0, HIST, init_carry=(zero16, zero16), unroll=5)
            def _sum(l, carry):
                a0, a1 = carry
                r = f * HIST + l
                a0 = a0 + rows[r, pl.ds(0, HALF)]
                a1 = a1 + rows[r, pl.ds(HALF, HALF)]
                return a0, a1

            a0, a1 = _sum
            outv[pl.ds(f * DIM, HALF)] = a0 * rf
            outv[pl.ds(f * DIM + HALF, HALF)] = a1 * rf

        pltpu.sync_copy(outv, out_hbm.at[bb])


@jax.jit
def kernel(x, tables):
    tabf = tables.reshape(NUM_FIELDS * VOCAB, DIM)

    call = pl.kernel(
        _body,
        out_type=jax.ShapeDtypeStruct((BATCH, OUT_D), jnp.float32),
        mesh=plsc.VectorSubcoreMesh(core_axis_name="c", subcore_axis_name="s"),
        compiler_params=pltpu.CompilerParams(
            needs_layout_passes=False, use_tc_tiling_on_sc=False),
        scratch_types=[
            pltpu.VMEM((NUM_FIELDS, HIST), jnp.int32),
            pltpu.VMEM((IDS_PAD,), jnp.int32),
            pltpu.VMEM((IDS_PAD, DIM), jnp.float32),
            pltpu.VMEM((NUM_FIELDS, LANES), jnp.float32),
            pltpu.VMEM((OUT_D,), jnp.float32),
            pltpu.SemaphoreType.DMA,
        ],
    )
    return call(x, tabf)


# speedup vs baseline: 11.7578x; 1.0487x over previous
import functools

import numpy as np
import jax
import jax.numpy as jnp
from jax import lax
from jax.experimental import pallas as pl
from jax.experimental.pallas import tpu as pltpu
from jax.experimental.pallas import tpu_sc as plsc

NUM_FIELDS = 26
VOCAB = 100000
DIM = 32
BATCH = 4096
HIST = 50

LANES = 16
HIST_PAD = 56
NW = 32
B_PER_W = BATCH // NW
HALF = DIM // 2
OUT_D = NUM_FIELDS * DIM


def _body(x_hbm, tab_hbm, out_hbm,
          xv, gid2, rows, recip2, outv, sem):
    wid = lax.axis_index("s") * 2 + lax.axis_index("c")
    base = wid * B_PER_W
    zero16 = jnp.zeros((LANES,), jnp.float32)
    tail2 = lax.iota(jnp.int32, LANES) >= (LANES - 2)

    for f in range(NUM_FIELDS):
        gid2[f, pl.ds(HIST_PAD - LANES, LANES)] = jnp.zeros(
            (LANES,), jnp.int32)

    @pl.loop(0, B_PER_W)
    def _batch(b):
        bb = base + b
        pltpu.sync_copy(x_hbm.at[bb], xv)

        for f in range(NUM_FIELDS):
            xa = xv[f, pl.ds(0, LANES)]
            xb = xv[f, pl.ds(LANES, LANES)]
            xc = xv[f, pl.ds(2 * LANES, LANES)]
            xd = xv[f, pl.ds(HIST - LANES, LANES)]
            gid2[f, pl.ds(0, LANES)] = xa
            gid2[f, pl.ds(LANES, LANES)] = xb
            gid2[f, pl.ds(2 * LANES, LANES)] = xc
            gid2[f, pl.ds(HIST - LANES, LANES)] = xd
            cnt = (plsc.all_reduce_population_count(xa != 0)
                   + plsc.all_reduce_population_count(xb != 0)
                   + plsc.all_reduce_population_count(xc != 0)
                   + plsc.all_reduce_population_count((xd != 0) & tail2))
            recip2[f, pl.ds(0, LANES)] = 1.0 / jnp.maximum(
                cnt.astype(jnp.float32), 1.0)

        copies = [
            pltpu.async_copy(tab_hbm.at[f].at[gid2.at[f]], rows.at[f], sem)
            for f in range(NUM_FIELDS)
        ]
        for c in copies:
            c.wait()

        for f in range(NUM_FIELDS):
            rf = recip2[f, pl.ds(0, LANES)]

            @pl.loop(0, HIST, init_carry=(zero16, zero16), unroll=5)
            def _sum(l, carry):
                a0, a1 = carry
                a0 = a0 + rows[f, l, pl.ds(0, HALF)]
                a1 = a1 + rows[f, l, pl.ds(HALF, HALF)]
                return a0, a1

            a0, a1 = _sum
            outv[pl.ds(f * DIM, HALF)] = a0 * rf
            outv[pl.ds(f * DIM + HALF, HALF)] = a1 * rf

        pltpu.sync_copy(outv, out_hbm.at[bb])


@jax.jit
def kernel(x, tables):
    call = pl.kernel(
        _body,
        out_type=jax.ShapeDtypeStruct((BATCH, OUT_D), jnp.float32),
        mesh=plsc.VectorSubcoreMesh(core_axis_name="c", subcore_axis_name="s"),
        compiler_params=pltpu.CompilerParams(
            needs_layout_passes=False, use_tc_tiling_on_sc=False),
        scratch_types=[
            pltpu.VMEM((NUM_FIELDS, HIST), jnp.int32),
            pltpu.VMEM((NUM_FIELDS, HIST_PAD), jnp.int32),
            pltpu.VMEM((NUM_FIELDS, HIST_PAD, DIM), jnp.float32),
            pltpu.VMEM((NUM_FIELDS, LANES), jnp.float32),
            pltpu.VMEM((OUT_D,), jnp.float32),
            pltpu.SemaphoreType.DMA,
        ],
    )
    return call(x, tables)


# speedup vs baseline: 11.7657x; 1.0007x over previous
import functools

import numpy as np
import jax
import jax.numpy as jnp
from jax import lax
from jax.experimental import pallas as pl
from jax.experimental.pallas import tpu as pltpu
from jax.experimental.pallas import tpu_sc as plsc

NUM_FIELDS = 26
VOCAB = 100000
DIM = 32
BATCH = 4096
HIST = 50

LANES = 16
HIST_PAD = 56
NW = 32
B_PER_W = BATCH // NW
HALF = DIM // 2
OUT_D = NUM_FIELDS * DIM


def _body(x_hbm, tab_hbm, out_hbm,
          xv0, xv1, gid0, gid1, rows0, rows1, rcp0, rcp1, outv,
          sem0, sem1):
    wid = lax.axis_index("s") * 2 + lax.axis_index("c")
    base = wid * B_PER_W
    zero16 = jnp.zeros((LANES,), jnp.float32)
    tail2 = lax.iota(jnp.int32, LANES) >= (LANES - 2)

    def stage(b, xv, gid2, rcp, sem):
        pltpu.sync_copy(x_hbm.at[base + b], xv)
        for f in range(NUM_FIELDS):
            xa = xv[f, pl.ds(0, LANES)]
            xb = xv[f, pl.ds(LANES, LANES)]
            xc = xv[f, pl.ds(2 * LANES, LANES)]
            xd = xv[f, pl.ds(HIST - LANES, LANES)]
            gid2[f, pl.ds(0, LANES)] = xa
            gid2[f, pl.ds(LANES, LANES)] = xb
            gid2[f, pl.ds(2 * LANES, LANES)] = xc
            gid2[f, pl.ds(HIST - LANES, LANES)] = xd
            cnt = (plsc.all_reduce_population_count(xa != 0)
                   + plsc.all_reduce_population_count(xb != 0)
                   + plsc.all_reduce_population_count(xc != 0)
                   + plsc.all_reduce_population_count((xd != 0) & tail2))
            rcp[f, pl.ds(0, LANES)] = 1.0 / jnp.maximum(
                cnt.astype(jnp.float32), 1.0)
    def fire(gid2, rows, sem):
        for f in range(NUM_FIELDS):
            pltpu.async_copy(
                tab_hbm.at[f].at[gid2.at[f]],
                rows.at[f], sem)

    def drain_reduce(b, gid2, rows, rcp, sem):
        for f in range(NUM_FIELDS):
            pltpu.make_async_copy(
                tab_hbm.at[f].at[gid2.at[f]],
                rows.at[f], sem).wait()
        for f in range(NUM_FIELDS):
            rf = rcp[f, pl.ds(0, LANES)]

            @pl.loop(0, HIST, init_carry=(zero16, zero16), unroll=5)
            def _sum(l, carry):
                a0, a1 = carry
                a0 = a0 + rows[f, l, pl.ds(0, HALF)]
                a1 = a1 + rows[f, l, pl.ds(HALF, HALF)]
                return a0, a1

            a0, a1 = _sum
            outv[pl.ds(f * DIM, HALF)] = a0 * rf
            outv[pl.ds(f * DIM + HALF, HALF)] = a1 * rf
        pltpu.sync_copy(outv, out_hbm.at[base + b])

    def stage_full(b, xv, gid2, rows, rcp, sem):
        stage(b, xv, gid2, rcp, sem)
        fire(gid2, rows, sem)

    for g2 in (gid0, gid1):
        for f in range(NUM_FIELDS):
            g2[f, pl.ds(HIST_PAD - LANES, LANES)] = jnp.zeros(
                (LANES,), jnp.int32)

    stage_full(0, xv0, gid0, rows0, rcp0, sem0)

    @pl.loop(0, B_PER_W // 2 - 1)
    def _pair(t):
        b = 2 * t
        stage_full(b + 1, xv1, gid1, rows1, rcp1, sem1)
        drain_reduce(b, gid0, rows0, rcp0, sem0)
        stage_full(b + 2, xv0, gid0, rows0, rcp0, sem0)
        drain_reduce(b + 1, gid1, rows1, rcp1, sem1)

    stage_full(B_PER_W - 1, xv1, gid1, rows1, rcp1, sem1)
    drain_reduce(B_PER_W - 2, gid0, rows0, rcp0, sem0)
    drain_reduce(B_PER_W - 1, gid1, rows1, rcp1, sem1)


@jax.jit
def kernel(x, tables):
    call = pl.kernel(
        _body,
        out_type=jax.ShapeDtypeStruct((BATCH, OUT_D), jnp.float32),
        mesh=plsc.VectorSubcoreMesh(core_axis_name="c", subcore_axis_name="s"),
        compiler_params=pltpu.CompilerParams(
            needs_layout_passes=False, use_tc_tiling_on_sc=False),
        scratch_types=[
            pltpu.VMEM((NUM_FIELDS, HIST), jnp.int32),
            pltpu.VMEM((NUM_FIELDS, HIST), jnp.int32),
            pltpu.VMEM((NUM_FIELDS, HIST_PAD), jnp.int32),
            pltpu.VMEM((NUM_FIELDS, HIST_PAD), jnp.int32),
            pltpu.VMEM((NUM_FIELDS, HIST_PAD, DIM), jnp.float32),
            pltpu.VMEM((NUM_FIELDS, HIST_PAD, DIM), jnp.float32),
            pltpu.VMEM((NUM_FIELDS, LANES), jnp.float32),
            pltpu.VMEM((NUM_FIELDS, LANES), jnp.float32),
            pltpu.VMEM((OUT_D,), jnp.float32),
            pltpu.SemaphoreType.DMA,
            pltpu.SemaphoreType.DMA,
        ],
    )
    return call(x, tables)
